# baseline (speedup 1.0000x reference)
import functools

import jax
import jax.numpy as jnp
import numpy as np
from jax.experimental import pallas as pl
from jax.experimental.pallas import tpu as pltpu

_SIGMA_TH = 0.004
_N_BINS = 1000
_N_L = 1001
_N = 128
_PAD = 1024
_HALF_BW = 0.0015707963611930609


@functools.lru_cache(maxsize=1)
def _consts():
    with jax.ensure_compile_time_eval():
        return _consts_impl()


def _consts_impl():
    n_bins = _N_BINS
    n_L = _N_L
    bin_width = jnp.pi / n_bins
    bins = jnp.linspace(0.0, jnp.pi, n_bins + 1)[:-1] + bin_width / 2
    ls = jnp.arange(n_L, dtype=jnp.float32)
    c0 = (1.0 - jnp.cos(bins)) / jnp.pi
    c2 = (2.0 * ls + 1.0)[None, :] * jnp.sin(
        (ls + 0.5)[None, :] * bins[:, None]
    ) / jnp.sin(bins[:, None] / 2.0)

    g = jax.random.gumbel(jax.random.key(1), (_N, n_bins), jnp.float32)
    u = jax.random.uniform(jax.random.key(2), (_N,), dtype=jnp.float32)
    delta = bin_width * (u - 0.5)
    nrm = jax.random.normal(jax.random.key(3), (_N,), dtype=jnp.float32)

    c2t = np.zeros((_PAD, _PAD), np.float32)
    c2t[:n_L, :n_bins] = np.asarray(c2).T
    lsq_neg = np.zeros((1, _PAD), np.float32)
    lsq_neg[0, :n_L] = np.asarray(-ls * (ls + 1.0))
    gz = np.full((_N, _PAD), -np.inf, np.float32)
    gz[:, :n_bins] = (
        np.asarray(g, np.float64) + np.log(np.asarray(c0, np.float64))
    ).astype(np.float32)
    dn = np.stack(
        [np.asarray(delta), np.asarray(nrm)], axis=1
    ).astype(np.float32)
    return c2t, lsq_neg, gz, dn


_NK = 2
_KC = _PAD // _NK


def _body(sigma_ref, lsq_neg_ref, c2t_hbm, gz_ref, dn_ref, out_ref,
          buf0, buf1, sem0, sem1):
    cp0 = pltpu.make_async_copy(c2t_hbm.at[pl.ds(0, _KC), :], buf0, sem0)
    cp1 = pltpu.make_async_copy(c2t_hbm.at[pl.ds(_KC, _KC), :], buf1, sem1)
    cp0.start()
    cp1.start()
    sig = sigma_ref[:, :]
    sig2 = sig * sig
    c1 = jnp.exp(lsq_neg_ref[:, :] * sig2)
    cp0.wait()
    s = jax.lax.dot_general(
        c1[:, :_KC],
        buf0[:, :],
        (((1,), (0,)), ((), ())),
        precision=jax.lax.Precision.HIGHEST,
        preferred_element_type=jnp.float32,
    )
    cp1.wait()
    s = s + jax.lax.dot_general(
        c1[:, _KC:],
        buf1[:, :],
        (((1,), (0,)), ((), ())),
        precision=jax.lax.Precision.HIGHEST,
        preferred_element_type=jnp.float32,
    )
    s = jnp.maximum(s, 0.0)
    t = jnp.log(s) + gz_ref[:, :]
    tmax = jnp.max(t, axis=1, keepdims=True)
    iota = jax.lax.broadcasted_iota(jnp.int32, (_N, _PAD), 1)
    idx = jnp.min(jnp.where(t == tmax, iota, 1 << 30), axis=1, keepdims=True)
    bv = np.float32(np.pi) * (idx.astype(jnp.float32) / _N_BINS) + _HALF_BW
    angle = bv + dn_ref[:, 0:1]
    gauss = sig * 2.0 + dn_ref[:, 1:2] * sig
    out_ref[:, :] = jnp.where(sig < _SIGMA_TH, gauss, angle)


def kernel(sigma):
    c2t, lsq_neg, gz, dn = _consts()
    out = pl.pallas_call(
        _body,
        in_specs=[
            pl.BlockSpec(memory_space=pltpu.VMEM),
            pl.BlockSpec(memory_space=pltpu.VMEM),
            pl.BlockSpec(memory_space=pl.ANY),
            pl.BlockSpec(memory_space=pltpu.VMEM),
            pl.BlockSpec(memory_space=pltpu.VMEM),
        ],
        out_specs=pl.BlockSpec(memory_space=pltpu.VMEM),
        out_shape=jax.ShapeDtypeStruct((_N, 1), jnp.float32),
        scratch_shapes=[
            pltpu.VMEM((_KC, _PAD), jnp.float32),
            pltpu.VMEM((_KC, _PAD), jnp.float32),
            pltpu.SemaphoreType.DMA,
            pltpu.SemaphoreType.DMA,
        ],
    )(
        sigma.reshape(_N, 1),
        lsq_neg,
        c2t,
        gz,
        dn,
    )
    return out.reshape(_N)


# speedup vs baseline: 1.0192x; 1.0192x over previous
import functools

import jax
import jax.numpy as jnp
import numpy as np
from jax.experimental import pallas as pl
from jax.experimental.pallas import tpu as pltpu

_SIGMA_TH = 0.004
_N_BINS = 1000
_N_L = 1001
_N = 128
_PAD = 1024
_HALF_BW = 0.0015707963611930609


@functools.lru_cache(maxsize=1)
def _consts():
    with jax.ensure_compile_time_eval():
        return _consts_impl()


def _consts_impl():
    n_bins = _N_BINS
    n_L = _N_L
    bin_width = jnp.pi / n_bins
    bins = jnp.linspace(0.0, jnp.pi, n_bins + 1)[:-1] + bin_width / 2
    ls = jnp.arange(n_L, dtype=jnp.float32)
    c0 = (1.0 - jnp.cos(bins)) / jnp.pi
    c2 = (2.0 * ls + 1.0)[None, :] * jnp.sin(
        (ls + 0.5)[None, :] * bins[:, None]
    ) / jnp.sin(bins[:, None] / 2.0)

    g = jax.random.gumbel(jax.random.key(1), (_N, n_bins), jnp.float32)
    u = jax.random.uniform(jax.random.key(2), (_N,), dtype=jnp.float32)
    delta = bin_width * (u - 0.5)
    nrm = jax.random.normal(jax.random.key(3), (_N,), dtype=jnp.float32)

    c2t = np.zeros((_PAD, _PAD), np.float32)
    c2t[:n_L, :n_bins] = np.asarray(c2).T
    lsq_neg = np.zeros((1, _PAD), np.float32)
    lsq_neg[0, :n_L] = np.asarray(-ls * (ls + 1.0))
    gz = np.full((_N, _PAD), -np.inf, np.float32)
    gz[:, :n_bins] = (
        np.asarray(g, np.float64) + np.log(np.asarray(c0, np.float64))
    ).astype(np.float32)
    dn = np.stack(
        [np.asarray(delta), np.asarray(nrm)], axis=1
    ).astype(np.float32)
    return c2t, lsq_neg, gz, dn


_NB = 2
_BC = _PAD // _NB


def _body(sigma_ref, lsq_neg_ref, c2t_ref, gz_ref, dn_ref, out_ref,
          c1_save, m_acc, i_acc):
    k = pl.program_id(0)
    sig = sigma_ref[:, :]

    @pl.when(k == 0)
    def _mk_c1():
        sig2 = sig * sig
        c1_save[:, :] = jnp.exp(lsq_neg_ref[:, :] * sig2)

    s = jax.lax.dot_general(
        c1_save[:, :],
        c2t_ref[:, :],
        (((1,), (0,)), ((), ())),
        precision=jax.lax.Precision.HIGHEST,
        preferred_element_type=jnp.float32,
    )
    s = jnp.maximum(s, 0.0)
    t = jnp.log(s) + gz_ref[:, :]
    tmax = jnp.max(t, axis=1, keepdims=True)
    iota = jax.lax.broadcasted_iota(jnp.int32, (_N, _BC), 1) + k * _BC
    idx = jnp.min(jnp.where(t == tmax, iota, 1 << 30), axis=1, keepdims=True)

    @pl.when(k == 0)
    def _init():
        m_acc[:, :] = tmax
        i_acc[:, :] = idx

    @pl.when(k > 0)
    def _combine():
        m_prev = m_acc[:, :]
        i_prev = i_acc[:, :]
        better = tmax > m_prev
        m_acc[:, :] = jnp.where(better, tmax, m_prev)
        i_acc[:, :] = jnp.where(better, idx, i_prev)

    @pl.when(k == _NB - 1)
    def _epilogue():
        fi = i_acc[:, :].astype(jnp.float32)
        bv = np.float32(np.pi) * (fi / _N_BINS) + _HALF_BW
        angle = bv + dn_ref[:, 0:1]
        gauss = sig * 2.0 + dn_ref[:, 1:2] * sig
        out_ref[:, :] = jnp.where(sig < _SIGMA_TH, gauss, angle)


def kernel(sigma):
    c2t, lsq_neg, gz, dn = _consts()
    out = pl.pallas_call(
        _body,
        grid=(_NB,),
        in_specs=[
            pl.BlockSpec((_N, 1), lambda k: (0, 0)),
            pl.BlockSpec((1, _PAD), lambda k: (0, 0)),
            pl.BlockSpec((_PAD, _BC), lambda k: (0, k)),
            pl.BlockSpec((_N, _BC), lambda k: (0, k)),
            pl.BlockSpec((_N, 2), lambda k: (0, 0)),
        ],
        out_specs=pl.BlockSpec((_N, 1), lambda k: (0, 0)),
        out_shape=jax.ShapeDtypeStruct((_N, 1), jnp.float32),
        scratch_shapes=[
            pltpu.VMEM((_N, _PAD), jnp.float32),
            pltpu.VMEM((_N, 1), jnp.float32),
            pltpu.VMEM((_N, 1), jnp.int32),
        ],
    )(
        sigma.reshape(_N, 1),
        lsq_neg,
        c2t,
        gz,
        dn,
    )
    return out.reshape(_N)


# speedup vs baseline: 1.1139x; 1.0928x over previous
import functools

import jax
import jax.numpy as jnp
import numpy as np
from jax.experimental import pallas as pl
from jax.experimental.pallas import tpu as pltpu

_SIGMA_TH = 0.004
_N_BINS = 1000
_N_L = 1001
_N = 128
_PAD = 1024
_HALF_BW = 0.0015707963611930609


@functools.lru_cache(maxsize=1)
def _consts():
    with jax.ensure_compile_time_eval():
        return _consts_impl()


def _consts_impl():
    n_bins = _N_BINS
    n_L = _N_L
    bin_width = jnp.pi / n_bins
    bins = jnp.linspace(0.0, jnp.pi, n_bins + 1)[:-1] + bin_width / 2
    ls = jnp.arange(n_L, dtype=jnp.float32)
    c0 = (1.0 - jnp.cos(bins)) / jnp.pi
    c2 = (2.0 * ls + 1.0)[None, :] * jnp.sin(
        (ls + 0.5)[None, :] * bins[:, None]
    ) / jnp.sin(bins[:, None] / 2.0)

    g = jax.random.gumbel(jax.random.key(1), (_N, n_bins), jnp.float32)
    u = jax.random.uniform(jax.random.key(2), (_N,), dtype=jnp.float32)
    delta = bin_width * (u - 0.5)
    nrm = jax.random.normal(jax.random.key(3), (_N,), dtype=jnp.float32)

    c2t = np.zeros((_PAD, _PAD), np.float32)
    c2t[:n_L, :n_bins] = np.asarray(c2).T
    lsq_neg = np.zeros((1, _PAD), np.float32)
    lsq_neg[0, :n_L] = np.asarray(-ls * (ls + 1.0))
    gz = np.full((_N, _PAD), -np.inf, np.float32)
    gz[:, :n_bins] = (
        np.asarray(g, np.float64) + np.log(np.asarray(c0, np.float64))
    ).astype(np.float32)
    dn = np.stack(
        [np.asarray(delta), np.asarray(nrm)], axis=1
    ).astype(np.float32)
    return c2t, lsq_neg, gz, dn


_NK = 2
_KC = _PAD // _NK


def _body(sigma_ref, lsq_neg_ref, c2t_ref, gz_ref, dn_ref, out_ref, s_acc):
    k = pl.program_id(0)
    sig = sigma_ref[:, :]
    sig2 = sig * sig
    c1 = jnp.exp(lsq_neg_ref[:, :] * sig2)
    partial = jax.lax.dot_general(
        c1,
        c2t_ref[:, :],
        (((1,), (0,)), ((), ())),
        precision=jax.lax.Precision.HIGHEST,
        preferred_element_type=jnp.float32,
    )

    @pl.when(k == 0)
    def _init():
        s_acc[:, :] = partial

    @pl.when(k > 0)
    def _accum():
        s_acc[:, :] = s_acc[:, :] + partial

    @pl.when(k == _NK - 1)
    def _epilogue():
        s = jnp.maximum(s_acc[:, :], 0.0)
        t = jnp.log(s) + gz_ref[:, :]
        tmax = jnp.max(t, axis=1, keepdims=True)
        iota = jax.lax.broadcasted_iota(jnp.int32, (_N, _PAD), 1)
        idx = jnp.min(jnp.where(t == tmax, iota, 1 << 30), axis=1,
                      keepdims=True)
        bv = np.float32(np.pi) * (idx.astype(jnp.float32) / _N_BINS) + _HALF_BW
        angle = bv + dn_ref[:, 0:1]
        gauss = sig * 2.0 + dn_ref[:, 1:2] * sig
        out_ref[:, :] = jnp.where(sig < _SIGMA_TH, gauss, angle)


def kernel(sigma):
    c2t, lsq_neg, gz, dn = _consts()
    out = pl.pallas_call(
        _body,
        grid=(_NK,),
        in_specs=[
            pl.BlockSpec((_N, 1), lambda k: (0, 0)),
            pl.BlockSpec((1, _KC), lambda k: (0, k)),
            pl.BlockSpec((_KC, _PAD), lambda k: (k, 0)),
            pl.BlockSpec((_N, _PAD), lambda k: (0, 0)),
            pl.BlockSpec((_N, 2), lambda k: (0, 0)),
        ],
        out_specs=pl.BlockSpec((_N, 1), lambda k: (0, 0)),
        out_shape=jax.ShapeDtypeStruct((_N, 1), jnp.float32),
        scratch_shapes=[pltpu.VMEM((_N, _PAD), jnp.float32)],
    )(
        sigma.reshape(_N, 1),
        lsq_neg,
        c2t,
        gz,
        dn,
    )
    return out.reshape(_N)


# speedup vs baseline: 1.1200x; 1.0055x over previous
import functools

import jax
import jax.numpy as jnp
import numpy as np
from jax.experimental import pallas as pl
from jax.experimental.pallas import tpu as pltpu

_SIGMA_TH = 0.004
_N_BINS = 1000
_N_L = 1001
_N = 128
_PAD = 1024
_HALF_BW = 0.0015707963611930609


@functools.lru_cache(maxsize=1)
def _consts():
    with jax.ensure_compile_time_eval():
        return _consts_impl()


def _consts_impl():
    n_bins = _N_BINS
    n_L = _N_L
    bin_width = jnp.pi / n_bins
    bins = jnp.linspace(0.0, jnp.pi, n_bins + 1)[:-1] + bin_width / 2
    ls = jnp.arange(n_L, dtype=jnp.float32)
    c0 = (1.0 - jnp.cos(bins)) / jnp.pi
    c2 = (2.0 * ls + 1.0)[None, :] * jnp.sin(
        (ls + 0.5)[None, :] * bins[:, None]
    ) / jnp.sin(bins[:, None] / 2.0)

    g = jax.random.gumbel(jax.random.key(1), (_N, n_bins), jnp.float32)
    u = jax.random.uniform(jax.random.key(2), (_N,), dtype=jnp.float32)
    delta = bin_width * (u - 0.5)
    nrm = jax.random.normal(jax.random.key(3), (_N,), dtype=jnp.float32)

    c2t = np.zeros((_PAD, _PAD), np.float32)
    c2t[:n_L, :n_bins] = np.asarray(c2).T
    lsq_neg = np.zeros((1, _PAD), np.float32)
    lsq_neg[0, :n_L] = np.asarray(-ls * (ls + 1.0))
    gz = np.full((_N, _PAD), -np.inf, np.float32)
    gz[:, :n_bins] = (
        np.asarray(g, np.float64) + np.log(np.asarray(c0, np.float64))
    ).astype(np.float32)
    dn = np.stack(
        [np.asarray(delta), np.asarray(nrm)], axis=1
    ).astype(np.float32)
    return c2t, lsq_neg, gz, dn


_NK = 2
_KC = _PAD // _NK


def _body(sigma_ref, lsq_neg_ref, c2t_ref, gz_ref, dn_ref, out_ref, s_acc):
    k = pl.program_id(0)
    sig = sigma_ref[:, :]
    sig2 = sig * sig
    c1 = jnp.exp(lsq_neg_ref[:, :] * sig2)
    partial = jax.lax.dot_general(
        c1,
        c2t_ref[:, :],
        (((1,), (0,)), ((), ())),
        precision=jax.lax.Precision.HIGHEST,
        preferred_element_type=jnp.float32,
    )

    @pl.when(k == 0)
    def _init():
        s_acc[:, :] = partial

    @pl.when((k > 0) & (k < _NK - 1))
    def _accum():
        s_acc[:, :] = s_acc[:, :] + partial

    @pl.when(k == _NK - 1)
    def _epilogue():
        s = jnp.maximum(s_acc[:, :] + partial, 0.0)
        t = jnp.log(s) + gz_ref[:, :]
        tmax = jnp.max(t, axis=1, keepdims=True)
        iota = jax.lax.broadcasted_iota(jnp.int32, (_N, _PAD), 1)
        idx = jnp.min(jnp.where(t == tmax, iota, 1 << 30), axis=1,
                      keepdims=True)
        bv = np.float32(np.pi) * (idx.astype(jnp.float32) / _N_BINS) + _HALF_BW
        angle = bv + dn_ref[:, 0:1]
        gauss = sig * 2.0 + dn_ref[:, 1:2] * sig
        out_ref[:, :] = jnp.where(sig < _SIGMA_TH, gauss, angle)


def kernel(sigma):
    c2t, lsq_neg, gz, dn = _consts()
    out = pl.pallas_call(
        _body,
        grid=(_NK,),
        in_specs=[
            pl.BlockSpec((_N, 1), lambda k: (0, 0)),
            pl.BlockSpec((1, _KC), lambda k: (0, k)),
            pl.BlockSpec((_KC, _PAD), lambda k: (k, 0)),
            pl.BlockSpec((_N, _PAD), lambda k: (0, 0)),
            pl.BlockSpec((_N, 2), lambda k: (0, 0)),
        ],
        out_specs=pl.BlockSpec((_N, 1), lambda k: (0, 0)),
        out_shape=jax.ShapeDtypeStruct((_N, 1), jnp.float32),
        scratch_shapes=[pltpu.VMEM((_N, _PAD), jnp.float32)],
    )(
        sigma.reshape(_N, 1),
        lsq_neg,
        c2t,
        gz,
        dn,
    )
    return out.reshape(_N)
